# baseline (speedup 1.0000x reference)
import functools

import jax
import jax.numpy as jnp
from jax import lax
from jax.experimental import pallas as pl
from jax.experimental.pallas import tpu as pltpu
from jax.experimental.pallas import tpu_sc as plsc

VOCAB = 1_000_000
EMBED_DIM = 64
BATCH = 4096
HIST = 200

_NUM_WORKERS = 32
_TOTAL = BATCH * HIST
_PER_WORKER = _TOTAL // _NUM_WORKERS
_CHUNK = 1024
_STREAM = 128
_K = _CHUNK // _STREAM
_NCHUNK = _PER_WORKER // _CHUNK
_IDX_ROWS = _TOTAL // _STREAM


@functools.cache
def _make_sc_gather():
    mesh = plsc.VectorSubcoreMesh(core_axis_name="c", subcore_axis_name="s")

    @functools.partial(
        pl.kernel,
        mesh=mesh,
        out_type=jax.ShapeDtypeStruct((_TOTAL, EMBED_DIM), jnp.float32),
        scratch_types=[
            pltpu.VMEM((_K, _STREAM), jnp.int32),
            pltpu.VMEM((_CHUNK, EMBED_DIM), jnp.float32),
            pltpu.SemaphoreType.DMA,
        ],
        compiler_params=pltpu.CompilerParams(use_tc_tiling_on_sc=False),
    )
    def gather_kernel(idx_hbm, table_hbm, out_hbm, idx_v, rows_v, sem):
        nc = lax.axis_size("c")
        wid = lax.axis_index("s") * nc + lax.axis_index("c")

        def body(g, carry):
            row_base = pl.multiple_of(wid * _PER_WORKER + g * _CHUNK, _CHUNK)
            irow_base = pl.multiple_of(
                (wid * _PER_WORKER + g * _CHUNK) // _STREAM, _K
            )
            pltpu.sync_copy(idx_hbm.at[pl.ds(irow_base, _K), :], idx_v)
            copies = []
            for b in range(_K):
                copies.append(
                    pltpu.async_copy(
                        table_hbm.at[idx_v.at[b]],
                        rows_v.at[pl.ds(b * _STREAM, _STREAM), :],
                        sem,
                    )
                )
            for c in copies:
                c.wait()
            pltpu.sync_copy(rows_v, out_hbm.at[pl.ds(row_base, _CHUNK), :])
            return carry

        lax.fori_loop(0, _NCHUNK, body, 0)

    return gather_kernel


def kernel(words, weight):
    idx = words.reshape(_IDX_ROWS, _STREAM).astype(jnp.int32)
    out = _make_sc_gather()(idx, weight)
    return out.reshape(BATCH, HIST, EMBED_DIM)


# speedup vs baseline: 1.0152x; 1.0152x over previous
import functools

import jax
import jax.numpy as jnp
from jax import lax
from jax.experimental import pallas as pl
from jax.experimental.pallas import tpu as pltpu
from jax.experimental.pallas import tpu_sc as plsc

VOCAB = 1_000_000
EMBED_DIM = 64
BATCH = 4096
HIST = 200

_NUM_WORKERS = 32
_TOTAL = BATCH * HIST
_PER_WORKER = _TOTAL // _NUM_WORKERS
_CHUNK = 512
_STREAM = 128
_K = _CHUNK // _STREAM
_NCHUNK = _PER_WORKER // _CHUNK
_IDX_ROWS_W = _PER_WORKER // _STREAM
_IDX_ROWS = _TOTAL // _STREAM
_NBUF = 2


def _make_sc_gather():
    mesh = plsc.VectorSubcoreMesh(core_axis_name="c", subcore_axis_name="s")

    @functools.partial(
        pl.kernel,
        mesh=mesh,
        out_type=jax.ShapeDtypeStruct((_TOTAL, EMBED_DIM), jnp.float32),
        scratch_types=[
            pltpu.VMEM((_IDX_ROWS_W, _STREAM), jnp.int32),
            pltpu.VMEM((_NBUF, _CHUNK, EMBED_DIM), jnp.float32),
            pltpu.SemaphoreType.DMA,
            pltpu.SemaphoreType.DMA,
        ],
        compiler_params=pltpu.CompilerParams(use_tc_tiling_on_sc=False),
    )
    def gather_kernel(idx_hbm, table_hbm, out_hbm, idx_v, rows_v, gsem, osem):
        nc = lax.axis_size("c")
        wid = lax.axis_index("s") * nc + lax.axis_index("c")
        w_row0 = pl.multiple_of(wid * _PER_WORKER, _PER_WORKER)
        w_irow0 = pl.multiple_of(wid * _IDX_ROWS_W, _IDX_ROWS_W)

        pltpu.sync_copy(idx_hbm.at[pl.ds(w_irow0, _IDX_ROWS_W), :], idx_v)

        def fire_gathers(c, buf):
            copies = []
            for b in range(_K):
                irow = c * _K + b
                copies.append(
                    pltpu.async_copy(
                        table_hbm.at[idx_v.at[irow]],
                        rows_v.at[buf, pl.ds(b * _STREAM, _STREAM), :],
                        gsem,
                    )
                )
            return copies

        def out_slice(c):
            base = pl.multiple_of(w_row0 + c * _CHUNK, _CHUNK)
            return out_hbm.at[pl.ds(base, _CHUNK), :]

        def fire_out(c, buf):
            return pltpu.async_copy(rows_v.at[buf], out_slice(c), osem)

        def wait_out_one(c, buf):
            pltpu.make_async_copy(rows_v.at[buf], out_slice(c), osem).wait()

        for c0 in range(_NBUF):
            for cp in fire_gathers(c0, c0):
                cp.wait()
            fire_out(c0, c0)

        def body(c, carry):
            buf = lax.rem(c, _NBUF)
            wait_out_one(c - _NBUF, buf)
            for cp in fire_gathers(c, buf):
                cp.wait()
            fire_out(c, buf)
            return carry

        lax.fori_loop(_NBUF, _NCHUNK, body, 0)

        for c0 in range(_NBUF):
            wait_out_one(_NCHUNK - _NBUF + c0, c0)

    return gather_kernel


@functools.cache
def _sc_gather():
    return _make_sc_gather()


def kernel(words, weight):
    idx = words.reshape(_IDX_ROWS, _STREAM).astype(jnp.int32)
    out = _sc_gather()(idx, weight)
    return out.reshape(BATCH, HIST, EMBED_DIM)


# speedup vs baseline: 1.0195x; 1.0042x over previous
import functools

import jax
import jax.numpy as jnp
from jax import lax
from jax.experimental import pallas as pl
from jax.experimental.pallas import tpu as pltpu
from jax.experimental.pallas import tpu_sc as plsc

VOCAB = 1_000_000
EMBED_DIM = 64
BATCH = 4096
HIST = 200

_NUM_WORKERS = 32
_TOTAL = BATCH * HIST
_PER_WORKER = _TOTAL // _NUM_WORKERS
_CHUNK = 128
_NCHUNK = _PER_WORKER // _CHUNK
_IDX_ROWS = _TOTAL // _CHUNK
_NBUF = 12
_DG = 8


def _make_sc_gather():
    mesh = plsc.VectorSubcoreMesh(core_axis_name="c", subcore_axis_name="s")

    @functools.partial(
        pl.kernel,
        mesh=mesh,
        out_type=jax.ShapeDtypeStruct((_TOTAL, EMBED_DIM), jnp.float32),
        scratch_types=[
            pltpu.VMEM((_NCHUNK, _CHUNK), jnp.int32),
            pltpu.VMEM((_NBUF, _CHUNK, EMBED_DIM), jnp.float32),
            pltpu.SemaphoreType.DMA,
            pltpu.SemaphoreType.DMA,
        ],
        compiler_params=pltpu.CompilerParams(use_tc_tiling_on_sc=False),
    )
    def gather_kernel(idx_hbm, table_hbm, out_hbm, idx_v, rows_v, gsem, osem):
        nc = lax.axis_size("c")
        wid = lax.axis_index("s") * nc + lax.axis_index("c")
        w_row0 = pl.multiple_of(wid * _PER_WORKER, _PER_WORKER)
        w_irow0 = pl.multiple_of(wid * _NCHUNK, _NCHUNK)

        pltpu.sync_copy(idx_hbm.at[pl.ds(w_irow0, _NCHUNK), :], idx_v)

        def buf_of(c):
            return lax.rem(c, _NBUF) if not isinstance(c, int) else c % _NBUF

        def gather_copy(c):
            return pltpu.make_async_copy(
                table_hbm.at[idx_v.at[c]], rows_v.at[buf_of(c)], gsem
            )

        def out_copy(c):
            base = pl.multiple_of(w_row0 + c * _CHUNK, _CHUNK)
            return pltpu.make_async_copy(
                rows_v.at[buf_of(c)], out_hbm.at[pl.ds(base, _CHUNK), :], osem
            )

        for t in range(_DG):
            gather_copy(t).start()
        for t in range(_DG, _NBUF):
            gather_copy(t - _DG).wait()
            out_copy(t - _DG).start()
            gather_copy(t).start()

        def body(t, carry):
            w = t - _DG
            gather_copy(w).wait()
            out_copy(w).start()
            out_copy(t - _NBUF).wait()
            gather_copy(t).start()
            return carry

        lax.fori_loop(_NBUF, _NCHUNK, body, 0)

        for t in range(_NCHUNK, _NCHUNK + _DG):
            gather_copy(t - _DG).wait()
            out_copy(t - _DG).start()
        for c in range(_NCHUNK - _NBUF, _NCHUNK):
            out_copy(c).wait()

    return gather_kernel


@functools.cache
def _sc_gather():
    return _make_sc_gather()


def kernel(words, weight):
    idx = words.reshape(_IDX_ROWS, _CHUNK).astype(jnp.int32)
    out = _sc_gather()(idx, weight)
    return out.reshape(BATCH, HIST, EMBED_DIM)
